# Initial kernel scaffold; baseline (speedup 1.0000x reference)
import jax
import jax.numpy as jnp
from jax.experimental import pallas as pl

N = 10000
E = 320000
DF = 128
DE = 16
H = 4
HID = 128
C = HID // H
L = 4
G = 64
TD = 32


def _time_enc(t):
    half = TD // 2
    fac = jnp.exp(jnp.arange(half, dtype=jnp.float32) * (-jnp.log(10000.0) / (half - 1 + 1e-08)))
    ang = t[:, None] * fac[None, :]
    return jnp.concatenate([jnp.sin(ang), jnp.cos(ang)], axis=-1)


def _layernorm(x, g, b):
    mu = jnp.mean(x, axis=-1, keepdims=True)
    v = jnp.mean((x - mu) ** 2, axis=-1, keepdims=True)
    return (x - mu) / jnp.sqrt(v + 1e-05) * g + b


def _inproj_body(xt_ref, w_ref, b_ref, o_ref):
    o_ref[...] = jax.nn.relu(xt_ref[...] @ w_ref[...] + b_ref[...])


def _inproj(xt, w, b):
    n = xt.shape[0]
    blk = 1024
    npad = ((n + blk - 1) // blk) * blk
    xt = jnp.pad(xt, ((0, npad - n), (0, 0)))
    out = pl.pallas_call(
        _inproj_body,
        grid=(npad // blk,),
        in_specs=[
            pl.BlockSpec((blk, xt.shape[1]), lambda i: (i, 0)),
            pl.BlockSpec((xt.shape[1], HID), lambda i: (0, 0)),
            pl.BlockSpec((1, HID), lambda i: (0, 0)),
        ],
        out_specs=pl.BlockSpec((blk, HID), lambda i: (i, 0)),
        out_shape=jax.ShapeDtypeStruct((npad, HID), jnp.float32),
    )(xt, w, b[None, :])
    return out[:n]


def _tconv(x, src, dst, ea, Wq, bq, Wk, bk, Wv, bv, We, Wskip, bskip, Wbeta):
    n = x.shape[0]
    q = (x @ Wq + bq).reshape(n, H, C)
    k = (x @ Wk + bk).reshape(n, H, C)
    v = (x @ Wv + bv).reshape(n, H, C)
    e = (ea @ We).reshape(-1, H, C)
    qi = q[dst]
    kj = k[src] + e
    vj = v[src] + e
    alpha = jnp.sum(qi * kj, axis=-1) / jnp.sqrt(float(C))
    amax = jax.ops.segment_max(alpha, dst, num_segments=n)
    amax = jnp.where(jnp.isfinite(amax), amax, 0.0)
    ex = jnp.exp(alpha - amax[dst])
    den = jax.ops.segment_sum(ex, dst, num_segments=n)
    a = ex / (den[dst] + 1e-16)
    out = jax.ops.segment_sum(vj * a[:, :, None], dst, num_segments=n).reshape(n, H * C)
    xr = x @ Wskip + bskip
    beta = jax.nn.sigmoid(jnp.concatenate([out, xr, out - xr], axis=-1) @ Wbeta)
    return beta * xr + (1.0 - beta) * out


def kernel(x, t, edge_attr_s, W_in, b_in, Wq, bq, Wk, bk, Wv, bv, We, Wskip, bskip, Wbeta, ln1_g, ln1_b, ff_W1, ff_b1, ff_W2, ff_b2, ln2_g, ln2_b, head_W1, head_b1, head_W2, head_b2, edge_index_s, batch):
    src = edge_index_s[0]
    dst = edge_index_s[1]
    h = _inproj(jnp.concatenate([x, _time_enc(t)], axis=-1), W_in, b_in)
    for i in range(L):
        xs = _tconv(h, src, dst, edge_attr_s, Wq[i], bq[i], Wk[i], bk[i], Wv[i], bv[i], We[i], Wskip[i], bskip[i], Wbeta[i])
        h = _layernorm(h + xs, ln1_g[i], ln1_b[i])
        ff = jax.nn.relu(h @ ff_W1[i] + ff_b1[i]) @ ff_W2[i] + ff_b2[i]
        h = _layernorm(h + ff, ln2_g[i], ln2_b[i])
    tmax = jax.ops.segment_max(t, batch, num_segments=G)
    m = (t == tmax[batch]).astype(h.dtype)
    cnt = jax.ops.segment_sum(m, batch, num_segments=G)
    gmean = jax.ops.segment_sum(h * m[:, None], batch, num_segments=G) / jnp.maximum(cnt, 1.0)[:, None]
    hm = jnp.where(m[:, None] > 0, h, -1e30)
    gmax = jax.ops.segment_max(hm, batch, num_segments=G)
    gmax = jnp.where(cnt[:, None] > 0, gmax, 0.0)
    g = jnp.concatenate([gmean, gmax], axis=-1)
    return jax.nn.relu(g @ head_W1 + head_b1) @ head_W2 + head_b2


# speedup vs baseline: 21.9444x; 21.9444x over previous
import functools
import math

import jax
import jax.numpy as jnp
from jax import lax
from jax.experimental import pallas as pl
from jax.experimental.pallas import tpu as pltpu
from jax.experimental.pallas import tpu_sc as plsc

N = 10000
NP = 10240
E = 320000
DF = 128
DE = 16
H = 4
HID = 128
C = HID // H
L = 4
G = 64
TD = 32

NC = 2
NS = 16
NW = NC * NS
EPW = E // NW
CHUNK = 80
NCHUNK = EPW // CHUNK
ROWS_PER_TILE = NP // NS
BE = 2560
BN = 1024


def _inproj_body(xt_ref, w_ref, b_ref, o_ref):
    o_ref[...] = jax.nn.relu(xt_ref[...] @ w_ref[...] + b_ref[0:1, :])


def _inproj(xt, w, b):
    return pl.pallas_call(
        _inproj_body,
        grid=(NP // BN,),
        in_specs=[
            pl.BlockSpec((BN, xt.shape[1]), lambda i: (i, 0)),
            pl.BlockSpec((xt.shape[1], HID), lambda i: (0, 0)),
            pl.BlockSpec((8, HID), lambda i: (0, 0)),
        ],
        out_specs=pl.BlockSpec((BN, HID), lambda i: (i, 0)),
        out_shape=jax.ShapeDtypeStruct((NP, HID), jnp.float32),
    )(xt, w, jnp.broadcast_to(b[None, :], (8, HID)))


def _qkv_body(h_ref, wq_ref, bq_ref, wk_ref, bk_ref, wv_ref, bv_ref,
              q_ref, k_ref, v_ref):
    hb = h_ref[...]
    q_ref[...] = hb @ wq_ref[...] + bq_ref[0:1, :]
    k_ref[...] = hb @ wk_ref[...] + bk_ref[0:1, :]
    v_ref[...] = hb @ wv_ref[...] + bv_ref[0:1, :]


def _qkv(h, Wq, bq, Wk, bk, Wv, bv):
    w_spec = pl.BlockSpec((HID, HID), lambda i: (0, 0))
    b_spec = pl.BlockSpec((8, HID), lambda i: (0, 0))
    n_spec = pl.BlockSpec((BN, HID), lambda i: (i, 0))
    bcast = lambda b: jnp.broadcast_to(b[None, :], (8, HID))
    return pl.pallas_call(
        _qkv_body,
        grid=(NP // BN,),
        in_specs=[n_spec, w_spec, b_spec, w_spec, b_spec, w_spec, b_spec],
        out_specs=[n_spec, n_spec, n_spec],
        out_shape=[jax.ShapeDtypeStruct((NP, HID), jnp.float32)] * 3,
    )(h, Wq, bcast(bq), Wk, bcast(bk), Wv, bcast(bv))


def _alpha_body(qi_ref, kj_ref, ea_ref, we_ref, a_ref):
    p = qi_ref[...] * (kj_ref[...] + ea_ref[...] @ we_ref[...])
    scale = 1.0 / math.sqrt(float(C))
    cols = [jnp.sum(p[:, h * C:(h + 1) * C], axis=-1, keepdims=True) * scale
            for h in range(H)]
    a_ref[...] = jnp.concatenate(cols + cols, axis=-1)


def _alpha(qi, kj, ea, We):
    return pl.pallas_call(
        _alpha_body,
        grid=(E // BE,),
        in_specs=[
            pl.BlockSpec((BE, HID), lambda i: (i, 0)),
            pl.BlockSpec((BE, HID), lambda i: (i, 0)),
            pl.BlockSpec((BE, DE), lambda i: (i, 0)),
            pl.BlockSpec((DE, HID), lambda i: (0, 0)),
        ],
        out_specs=pl.BlockSpec((BE, 2 * H), lambda i: (i, 0)),
        out_shape=jax.ShapeDtypeStruct((E, 2 * H), jnp.float32),
    )(qi, kj, ea, We)


def _wrow_body(vj_ref, ea_ref, we_ref, a_ref, gmax_ref, m_ref, x_ref):
    e = ea_ref[...] @ we_ref[...]
    vje = vj_ref[...] + e
    a = a_ref[...]
    gm = gmax_ref[...]
    ex_cols = [jnp.exp(a[:, h:h + 1] - gm[0:1, h:h + 1]) for h in range(H)]
    exfull = jnp.concatenate(
        [jnp.broadcast_to(ex_cols[h], (vje.shape[0], C)) for h in range(H)],
        axis=-1)
    m_ref[...] = vje * exfull
    x_ref[...] = exfull


def _wrow(vj, ea, We, alpha, gmax8):
    return pl.pallas_call(
        _wrow_body,
        grid=(E // BE,),
        in_specs=[
            pl.BlockSpec((BE, HID), lambda i: (i, 0)),
            pl.BlockSpec((BE, DE), lambda i: (i, 0)),
            pl.BlockSpec((DE, HID), lambda i: (0, 0)),
            pl.BlockSpec((BE, 2 * H), lambda i: (i, 0)),
            pl.BlockSpec((8, 2 * H), lambda i: (0, 0)),
        ],
        out_specs=[pl.BlockSpec((BE, HID), lambda i: (i, 0))] * 2,
        out_shape=[jax.ShapeDtypeStruct((E, HID), jnp.float32)] * 2,
    )(vj, ea, We, alpha, gmax8)


def _sigmoid(x):
    return 1.0 / (1.0 + jnp.exp(-x))


def _ln(x, g, b):
    mu = jnp.mean(x, axis=-1, keepdims=True)
    xc = x - mu
    v = jnp.mean(xc * xc, axis=-1, keepdims=True)
    return xc * jax.lax.rsqrt(v + 1e-05) * g + b


def _node_body(accm0_ref, accm1_ref, accd0_ref, accd1_ref, h_ref,
               wskip_ref, bskip_ref, wbT_ref,
               ln1g_ref, ln1b_ref, ffw1_ref, ffb1_ref, ffw2_ref, ffb2_ref,
               ln2g_ref, ln2b_ref, o_ref):
    acc = accm0_ref[...] + accm1_ref[...]
    den = accd0_ref[...] + accd1_ref[...]
    out = acc / (den + 1e-16)
    hb = h_ref[...]
    xr = hb @ wskip_ref[...] + bskip_ref[0:1, :]
    wbT = wbT_ref[...]
    cat = jnp.concatenate([out, xr, out - xr], axis=-1)
    logit = jnp.sum(cat * wbT[0:1, :], axis=-1, keepdims=True)
    beta = _sigmoid(logit)
    xs = beta * xr + (1.0 - beta) * out
    h1 = _ln(hb + xs, ln1g_ref[0:1, :], ln1b_ref[0:1, :])
    ff = jax.nn.relu(h1 @ ffw1_ref[...] + ffb1_ref[0:1, :]) @ ffw2_ref[...] + ffb2_ref[0:1, :]
    o_ref[...] = _ln(h1 + ff, ln2g_ref[0:1, :], ln2b_ref[0:1, :])


def _node(accm0, accm1, accd0, accd1, h, Wskip, bskip, Wbeta, ln1g, ln1b,
          ffW1, ffb1, ffW2, ffb2, ln2g, ln2b):
    n_spec = pl.BlockSpec((BN, HID), lambda i: (i, 0))
    acc_spec = pl.BlockSpec((BN, HID), lambda i: (i, 0))
    b128 = pl.BlockSpec((8, HID), lambda i: (0, 0))
    bcast = lambda b: jnp.broadcast_to(b[None, :], (8, HID))
    wbT = jnp.broadcast_to(Wbeta[:, 0][None, :], (8, 3 * HID))
    return pl.pallas_call(
        _node_body,
        grid=(NP // BN,),
        in_specs=[
            acc_spec, acc_spec, acc_spec, acc_spec, n_spec,
            pl.BlockSpec((HID, HID), lambda i: (0, 0)), b128,
            pl.BlockSpec((8, 3 * HID), lambda i: (0, 0)),
            b128, b128,
            pl.BlockSpec((HID, 4 * HID), lambda i: (0, 0)),
            pl.BlockSpec((8, 4 * HID), lambda i: (0, 0)),
            pl.BlockSpec((4 * HID, HID), lambda i: (0, 0)), b128,
            b128, b128,
        ],
        out_specs=n_spec,
        out_shape=jax.ShapeDtypeStruct((NP, HID), jnp.float32),
    )(accm0, accm1, accd0, accd1, h, Wskip, bcast(bskip), wbT, bcast(ln1g), bcast(ln1b),
      ffW1, jnp.broadcast_to(ffb1[None, :], (8, 4 * HID)), ffW2, bcast(ffb2),
      bcast(ln2g), bcast(ln2b))


def _sc_gather_body(q_hbm, k_hbm, v_hbm, src_hbm, dst_hbm,
                    qi_hbm, kj_hbm, vj_hbm,
                    idx_s, idx_d, rq, rk, rv, sem):
    wid = lax.axis_index("s") * NC + lax.axis_index("c")
    base = wid * EPW

    def chunk(i, _):
        off = base + i * CHUNK
        pltpu.sync_copy(src_hbm.at[pl.ds(off, CHUNK)], idx_s)
        pltpu.sync_copy(dst_hbm.at[pl.ds(off, CHUNK)], idx_d)
        cq = pltpu.async_copy(q_hbm.at[idx_d], rq, sem)
        ck = pltpu.async_copy(k_hbm.at[idx_s], rk, sem)
        cv = pltpu.async_copy(v_hbm.at[idx_s], rv, sem)
        cq.wait()
        ck.wait()
        cv.wait()
        pltpu.sync_copy(rq, qi_hbm.at[pl.ds(off, CHUNK)])
        pltpu.sync_copy(rk, kj_hbm.at[pl.ds(off, CHUNK)])
        pltpu.sync_copy(rv, vj_hbm.at[pl.ds(off, CHUNK)])
        return ()

    lax.fori_loop(0, NCHUNK, chunk, ())


@functools.lru_cache(maxsize=None)
def _sc_kernels():
    mesh = plsc.VectorSubcoreMesh(core_axis_name="c", subcore_axis_name="s")
    gather = pl.kernel(
        _sc_gather_body,
        out_type=[jax.ShapeDtypeStruct((E, HID), jnp.float32)] * 3,
        mesh=mesh,
        scratch_types=[
            pltpu.VMEM((CHUNK,), jnp.int32),
            pltpu.VMEM((CHUNK,), jnp.int32),
            pltpu.VMEM((CHUNK, HID), jnp.float32),
            pltpu.VMEM((CHUNK, HID), jnp.float32),
            pltpu.VMEM((CHUNK, HID), jnp.float32),
            pltpu.SemaphoreType.DMA,
        ],
    )
    scatter = pl.kernel(
        _sc_scatter_body,
        out_type=[jax.ShapeDtypeStruct((NP, HID), jnp.float32)] * 2,
        mesh=mesh,
        scratch_types=[
            pltpu.VMEM((CHUNK,), jnp.int32),
            pltpu.VMEM((CHUNK, HID), jnp.float32),
            pltpu.VMEM_SHARED((NP, HID), jnp.float32),
        ],
    )
    return gather, scatter


def _sc_gather(q, k, v, src, dst):
    return _sc_kernels()[0](q, k, v, src, dst)


def _sc_scatter_body(w_hbm, dst_hbm, z_hbm, acc0_hbm, acc1_hbm,
                     idx, rows, accsh):
    c = lax.axis_index("c")
    s = lax.axis_index("s")
    rbase = s * ROWS_PER_TILE
    pltpu.sync_copy(z_hbm.at[pl.ds(rbase, ROWS_PER_TILE)],
                    accsh.at[pl.ds(rbase, ROWS_PER_TILE)])
    plsc.subcore_barrier()

    base = (c * NS + s) * EPW

    def chunk(i, _):
        off = base + i * CHUNK
        pltpu.sync_copy(dst_hbm.at[pl.ds(off, CHUNK)], idx)
        pltpu.sync_copy(w_hbm.at[pl.ds(off, CHUNK)], rows)
        pltpu.sync_copy(rows, accsh.at[idx], add=True)
        return ()

    lax.fori_loop(0, NCHUNK, chunk, ())
    plsc.subcore_barrier()

    @pl.when(c == 0)
    def _():
        pltpu.sync_copy(accsh.at[pl.ds(rbase, ROWS_PER_TILE)],
                        acc0_hbm.at[pl.ds(rbase, ROWS_PER_TILE)])

    @pl.when(c == 1)
    def _():
        pltpu.sync_copy(accsh.at[pl.ds(rbase, ROWS_PER_TILE)],
                        acc1_hbm.at[pl.ds(rbase, ROWS_PER_TILE)])


def _sc_scatter(w, dst, zrow):
    return _sc_kernels()[1](w, dst, zrow)


def _time_enc(t):
    half = TD // 2
    fac = jnp.exp(jnp.arange(half, dtype=jnp.float32) * (-jnp.log(10000.0) / (half - 1 + 1e-08)))
    ang = t[:, None] * fac[None, :]
    return jnp.concatenate([jnp.sin(ang), jnp.cos(ang)], axis=-1)


def _tconv(h, src, dst, ea, zrow, Wq, bq, Wk, bk, Wv, bv, We, Wskip, bskip,
           Wbeta, ln1g, ln1b, ffW1, ffb1, ffW2, ffb2, ln2g, ln2b):
    q, k, v = _qkv(h, Wq, bq, Wk, bk, Wv, bv)
    qi, kj, vj = _sc_gather(q, k, v, src, dst)
    alpha = _alpha(qi, kj, ea, We)
    gmax = jnp.max(alpha, axis=0)
    gmax8 = jnp.broadcast_to(gmax[None, :], (8, 2 * H))
    msg, exf = _wrow(vj, ea, We, alpha, gmax8)
    accm0, accm1 = _sc_scatter(msg, dst, zrow)
    accd0, accd1 = _sc_scatter(exf, dst, zrow)
    return _node(accm0, accm1, accd0, accd1, h, Wskip, bskip, Wbeta,
                 ln1g, ln1b, ffW1, ffb1, ffW2, ffb2, ln2g, ln2b)


def kernel(x, t, edge_attr_s, W_in, b_in, Wq, bq, Wk, bk, Wv, bv, We, Wskip, bskip, Wbeta, ln1_g, ln1_b, ff_W1, ff_b1, ff_W2, ff_b2, ln2_g, ln2_b, head_W1, head_b1, head_W2, head_b2, edge_index_s, batch):
    src = edge_index_s[0]
    dst = edge_index_s[1]
    xt = jnp.concatenate([x, _time_enc(t)], axis=-1)
    xt = jnp.pad(xt, ((0, NP - N), (0, 0)))
    h = _inproj(xt, W_in, b_in)
    zrow = jnp.zeros((NP, HID), jnp.float32)
    for i in range(L):
        h = _tconv(h, src, dst, edge_attr_s, zrow,
                   Wq[i], bq[i], Wk[i], bk[i], Wv[i], bv[i], We[i],
                   Wskip[i], bskip[i], Wbeta[i], ln1_g[i], ln1_b[i],
                   ff_W1[i], ff_b1[i], ff_W2[i], ff_b2[i], ln2_g[i], ln2_b[i])
    h = h[:N]
    tmax = jax.ops.segment_max(t, batch, num_segments=G)
    m = (t == tmax[batch]).astype(h.dtype)
    cnt = jax.ops.segment_sum(m, batch, num_segments=G)
    gmean = jax.ops.segment_sum(h * m[:, None], batch, num_segments=G) / jnp.maximum(cnt, 1.0)[:, None]
    hm = jnp.where(m[:, None] > 0, h, -1e30)
    gmax = jax.ops.segment_max(hm, batch, num_segments=G)
    gmax = jnp.where(cnt[:, None] > 0, gmax, 0.0)
    g = jnp.concatenate([gmean, gmax], axis=-1)
    return jax.nn.relu(g @ head_W1 + head_b1) @ head_W2 + head_b2
